# Initial kernel scaffold; baseline (speedup 1.0000x reference)
import jax
import jax.numpy as jnp
from jax.experimental import pallas as pl


def kernel(x, weight):
    raise NotImplementedError("write your pallas kernel here")


# speedup vs baseline: 1.5626x; 1.5626x over previous
import functools

import jax
import jax.numpy as jnp
from jax import lax
from jax.experimental import pallas as pl
from jax.experimental.pallas import tpu as pltpu
from jax.experimental.pallas import tpu_sc as plsc

_VOCAB = 1000000
_DIM = 32
_BATCH = 16384
_FIELDS = 26
_TOTAL = _BATCH * _FIELDS


@functools.lru_cache(maxsize=None)
def _build(total, dim):
    info = plsc.get_sparse_core_info()
    nc, ns = info.num_cores, info.num_subcores
    nw = nc * ns
    per_w = total // nw
    chunk = 1664
    n_chunks = per_w // chunk
    assert per_w % chunk == 0

    mesh = plsc.VectorSubcoreMesh(core_axis_name="c", subcore_axis_name="s")

    @functools.partial(
        pl.kernel,
        mesh=mesh,
        out_type=jax.ShapeDtypeStruct((total, dim), jnp.float32),
        scratch_types=[
            pltpu.VMEM((chunk,), jnp.int32),
            pltpu.VMEM((chunk, dim), jnp.float32),
            pltpu.SemaphoreType.DMA,
        ],
        compiler_params=pltpu.CompilerParams(use_tc_tiling_on_sc=False),
    )
    def gather_kernel(idx_hbm, table_hbm, out_hbm, idx_v, rows_v, sem):
        wid = lax.axis_index("s") * nc + lax.axis_index("c")
        base = wid * per_w

        def body(i, carry):
            off = base + i * chunk
            pltpu.sync_copy(idx_hbm.at[pl.ds(off, chunk)], idx_v)
            pltpu.async_copy(table_hbm.at[idx_v], rows_v, sem).wait()
            pltpu.sync_copy(rows_v, out_hbm.at[pl.ds(off, chunk)])
            return carry

        lax.fori_loop(0, n_chunks, body, 0)

    return gather_kernel


def kernel(x, weight):
    idx = x.reshape(-1).astype(jnp.int32)
    out = _build(_TOTAL, _DIM)(idx, weight)
    return out.reshape(x.shape + (weight.shape[1],))


# speedup vs baseline: 1.5756x; 1.0083x over previous
import functools

import jax
import jax.numpy as jnp
from jax import lax
from jax.experimental import pallas as pl
from jax.experimental.pallas import tpu as pltpu
from jax.experimental.pallas import tpu_sc as plsc

_DIM = 32
_BATCH = 16384
_FIELDS = 26
_TOTAL = _BATCH * _FIELDS
_CHUNK = 1024
_NBUF = 3


@functools.lru_cache(maxsize=None)
def _build(total, dim):
    info = plsc.get_sparse_core_info()
    nc, ns = info.num_cores, info.num_subcores
    nw = nc * ns
    per_w = total // nw
    chunk = _CHUNK
    nbuf = _NBUF
    n_chunks = per_w // chunk
    assert per_w % chunk == 0

    mesh = plsc.VectorSubcoreMesh(core_axis_name="c", subcore_axis_name="s")

    @functools.partial(
        pl.kernel,
        mesh=mesh,
        out_type=jax.ShapeDtypeStruct((total, dim), jnp.float32),
        scratch_types=[pltpu.VMEM((chunk,), jnp.int32)] * n_chunks
        + [pltpu.VMEM((chunk, dim), jnp.float32)] * nbuf
        + [pltpu.SemaphoreType.DMA] * (1 + 2 * nbuf),
        compiler_params=pltpu.CompilerParams(use_tc_tiling_on_sc=False),
    )
    def gather_kernel(idx_hbm, table_hbm, out_hbm, *rest):
        ibufs = rest[:n_chunks]
        rows = rest[n_chunks : n_chunks + nbuf]
        isem = rest[n_chunks + nbuf]
        gsems = rest[n_chunks + nbuf + 1 : n_chunks + 2 * nbuf + 1]
        ssems = rest[n_chunks + 2 * nbuf + 1 :]
        wid = lax.axis_index("s") * nc + lax.axis_index("c")
        base = wid * per_w

        def icopy(i):
            return pltpu.make_async_copy(
                idx_hbm.at[pl.ds(base + i * chunk, chunk)], ibufs[i], isem
            )

        def gcopy(i, b):
            return pltpu.make_async_copy(
                table_hbm.at[ibufs[i]], rows[b], gsems[b]
            )

        def scopy(i, b):
            return pltpu.make_async_copy(
                rows[b], out_hbm.at[pl.ds(base + i * chunk, chunk)], ssems[b]
            )

        for i in range(n_chunks):
            icopy(i).start()
        for i in range(n_chunks):
            icopy(i).wait()

        for b in range(min(nbuf, n_chunks)):
            gcopy(b, b).start()
        for i in range(n_chunks):
            b = i % nbuf
            gcopy(i, b).wait()
            scopy(i, b).start()
            nxt = i + nbuf
            if nxt < n_chunks:
                scopy(i, b).wait()
                gcopy(nxt, b).start()
        for i in range(n_chunks - min(nbuf, n_chunks), n_chunks):
            scopy(i, i % nbuf).wait()

    return gather_kernel


def kernel(x, weight):
    idx = x.reshape(-1).astype(jnp.int32)
    out = _build(_TOTAL, _DIM)(idx, weight)
    return out.reshape(x.shape + (weight.shape[1],))
